# baseline (speedup 1.0000x reference)
import functools

import jax
import jax.numpy as jnp
from jax import lax
from jax.experimental import pallas as pl
from jax.experimental.pallas import tpu as pltpu
from jax.experimental.pallas import tpu_sc as plsc

N_V = 10000
N_E = 2500
N_P = 320000
D = 128
NC = 2
NS = 16
L = 16
VPAD = 10240
EPAD = 2560
RBLK = 1280
NCH = 160
P_T = NCH * 128
P_PAD = NS * P_T

_mesh = lambda: plsc.VectorSubcoreMesh(core_axis_name="c", subcore_axis_name="s")
_SC_PARAMS = pltpu.CompilerParams(needs_layout_passes=False)


def _sc_degrees(vflat, eflat):
    @functools.partial(
        pl.kernel,
        mesh=_mesh(),
        compiler_params=_SC_PARAMS,
        out_type=(jax.ShapeDtypeStruct((NC * NS * VPAD,), jnp.float32),
                  jax.ShapeDtypeStruct((NC * NS * EPAD,), jnp.float32)),
        scratch_types=[
            pltpu.VMEM((P_T,), jnp.int32),
            pltpu.VMEM((P_T,), jnp.int32),
            pltpu.VMEM((VPAD,), jnp.float32),
            pltpu.VMEM((EPAD,), jnp.float32),
        ],
    )
    def k(vidx_hbm, eidx_hbm, dv_out, de_out, vbuf, ebuf, dvh, deh):
        c = lax.axis_index("c")
        s = lax.axis_index("s")
        wid = c * NS + s
        zero = jnp.zeros((L,), jnp.float32)

        def zloop(ref, n):
            def zbody(j, carry):
                ref[pl.ds(j * L, L)] = zero
                return carry
            lax.fori_loop(0, n, zbody, 0)

        zloop(dvh, VPAD // L)
        zloop(deh, EPAD // L)
        base = c * P_PAD + s * P_T
        pltpu.sync_copy(vidx_hbm.at[pl.ds(base, P_T)], vbuf)
        pltpu.sync_copy(eidx_hbm.at[pl.ds(base, P_T)], ebuf)
        ones = jnp.ones((L,), jnp.float32)

        def body(i, carry):
            vi = vbuf[pl.ds(i * L, L)]
            ei = ebuf[pl.ds(i * L, L)]
            plsc.addupdate_scatter(dvh, [vi], ones)
            plsc.addupdate_scatter(deh, [ei], ones)
            return carry

        lax.fori_loop(0, P_T // L, body, 0)
        pltpu.sync_copy(dvh, dv_out.at[pl.ds(wid * VPAD, VPAD)])
        pltpu.sync_copy(deh, de_out.at[pl.ds(wid * EPAD, EPAD)])

    return k(vflat, eflat)


def _make_sc_pass(trows, arows, chunk, nsec):
    nch = P_T // chunk
    spc = nch // nsec
    ng = spc // 4
    rpt = arows // NS

    @functools.partial(
        pl.kernel,
        mesh=_mesh(),
        compiler_params=_SC_PARAMS,
        out_type=jax.ShapeDtypeStruct((NC, arows, D), jnp.float32),
        scratch_types=[
            pltpu.VMEM_SHARED((arows, D), jnp.float32),
            pltpu.VMEM((spc, chunk), jnp.int32),
            pltpu.VMEM((spc, chunk), jnp.int32),
            pltpu.VMEM((chunk, D), jnp.float32),
            pltpu.VMEM((chunk, D), jnp.float32),
            pltpu.VMEM((chunk, D), jnp.float32),
            pltpu.VMEM((chunk, D), jnp.float32),
            pltpu.VMEM((16, D), jnp.float32),
            pltpu.SemaphoreType.DMA,
            pltpu.SemaphoreType.DMA,
            pltpu.SemaphoreType.DMA,
            pltpu.SemaphoreType.DMA,
            pltpu.SemaphoreType.DMA,
            pltpu.SemaphoreType.DMA,
            pltpu.SemaphoreType.DMA,
            pltpu.SemaphoreType.DMA,
            pltpu.SemaphoreType.DMA,
        ],
    )
    def k(table_hbm, gidx_hbm, sidx_hbm, out_hbm,
          acc, gsec, ssec, r0b, r1b, r2b, r3b, zbuf,
          sg0, sg1, sg2, sg3, ss0, ss1, ss2, ss3, sz):
        rows = (r0b, r1b, r2b, r3b)
        semg = (sg0, sg1, sg2, sg3)
        sems = (ss0, ss1, ss2, ss3)
        c = lax.axis_index("c")
        s = lax.axis_index("s")
        wid = c * NS + s
        zero = jnp.zeros((L,), jnp.float32)
        for j in range(16):
            for kk in range(D // L):
                zbuf[j, pl.ds(kk * L, L)] = zero
        r0 = s * rpt

        def zfire(r, carry):
            pltpu.async_copy(zbuf, acc.at[pl.ds(r0 + r * 16, 16)], sz)
            return carry

        def zdrain(r, carry):
            pltpu.make_async_copy(zbuf, acc.at[pl.ds(r0 + r * 16, 16)], sz).wait()
            return carry

        lax.fori_loop(0, rpt // 16, zfire, 0)
        lax.fori_loop(0, rpt // 16, zdrain, 0)
        plsc.subcore_barrier()

        tbl = table_hbm.at[c]

        def fire_g(j, k):
            pltpu.async_copy(tbl.at[gsec.at[j]], rows[k], semg[k])

        def wait_g(j, k):
            pltpu.make_async_copy(tbl.at[gsec.at[j]], rows[k], semg[k]).wait()

        def fire_s(j, k):
            pltpu.async_copy(rows[k], acc.at[ssec.at[j]], sems[k], add=True)

        def wait_s(j, k):
            pltpu.make_async_copy(rows[k], acc.at[ssec.at[j]], sems[k]).wait()

        for sec in range(nsec):
            pltpu.sync_copy(gidx_hbm.at[wid, pl.ds(sec * spc, spc)], gsec)
            pltpu.sync_copy(sidx_hbm.at[wid, pl.ds(sec * spc, spc)], ssec)
            for k in range(4):
                fire_g(k, k)

            def step(jj, carry):
                j = 4 * jj
                for k in range(4):
                    wait_g(j + k, k)
                    fire_s(j + k, k)
                for k in range(4):
                    wait_s(j + k, k)
                    fire_g(j + 4 + k, k)
                return carry

            lax.fori_loop(0, ng - 1, step, 0)
            jl = 4 * (ng - 1)
            for k in range(4):
                wait_g(jl + k, k)
                fire_s(jl + k, k)
            for k in range(4):
                wait_s(jl + k, k)

        plsc.subcore_barrier()
        pltpu.sync_copy(acc.at[pl.ds(r0, rpt)], out_hbm.at[c, pl.ds(r0, rpt)])

    return k


_sc_pass_e = _make_sc_pass(VPAD, EPAD, 128, 1)
_sc_pass_v = _make_sc_pass(EPAD, VPAD, 64, 8)


def _mxu(a, b):
    return jnp.dot(a.astype(jnp.bfloat16), b.astype(jnp.bfloat16),
                   preferred_element_type=jnp.float32)


def _tc_deg(dvp, dep):
    def body(dvp_ref, dep_ref, dvs_ref, dei_ref):
        for c in range(NC):
            dv = jnp.sum(dvp_ref[c * NS:(c + 1) * NS, :], axis=0)
            de = jnp.sum(dep_ref[c * NS:(c + 1) * NS, :], axis=0)
            dvs_ref[c, :, 0] = jnp.where(dv > 0, 1.0 / jnp.sqrt(jnp.maximum(dv, 1e-12)), 0.0)
            dei_ref[c, :, 0] = jnp.where(de > 0, 1.0 / jnp.maximum(de, 1e-12), 0.0)

    return pl.pallas_call(
        body,
        out_shape=(jax.ShapeDtypeStruct((NC, VPAD, 1), jnp.float32),
                   jax.ShapeDtypeStruct((NC, EPAD, 1), jnp.float32)),
    )(dvp, dep)


def _tc_pre(Xp, W1, b1, dvs):
    def body(x_ref, w_ref, b_ref, d_ref, o_ref):
        x = x_ref[...]
        o_ref[0] = (_mxu(x, w_ref[0]) + b_ref[0]) * d_ref[0]

    grid = (NC, VPAD // RBLK)
    return pl.pallas_call(
        body,
        grid=grid,
        in_specs=[
            pl.BlockSpec((RBLK, D), lambda c, i: (i, 0)),
            pl.BlockSpec((1, D, D), lambda c, i: (c, 0, 0)),
            pl.BlockSpec((1, 1, D), lambda c, i: (c, 0, 0)),
            pl.BlockSpec((1, RBLK, 1), lambda c, i: (c, i, 0)),
        ],
        out_specs=pl.BlockSpec((1, RBLK, D), lambda c, i: (c, i, 0)),
        out_shape=jax.ShapeDtypeStruct((NC, VPAD, D), jnp.float32),
    )(Xp, W1, b1, dvs)


def _tc_mid(accE, dei):
    def body(a_ref, d_ref, o_ref):
        o_ref[...] = a_ref[...] * d_ref[...]

    return pl.pallas_call(
        body,
        grid=(NC,),
        in_specs=[
            pl.BlockSpec((1, EPAD, D), lambda c: (c, 0, 0)),
            pl.BlockSpec((1, EPAD, 1), lambda c: (c, 0, 0)),
        ],
        out_specs=pl.BlockSpec((1, EPAD, D), lambda c: (c, 0, 0)),
        out_shape=jax.ShapeDtypeStruct((NC, EPAD, D), jnp.float32),
    )(accE, dei)


def _tc_postpre(accV, dvs, W2, b2):
    def body(a_ref, d_ref, w_ref, b_ref, o_ref):
        t = jnp.maximum(a_ref[0] * d_ref[0], 0.0)
        o_ref[0] = (_mxu(t, w_ref[0]) + b_ref[0]) * d_ref[0]

    grid = (NC, VPAD // RBLK)
    return pl.pallas_call(
        body,
        grid=grid,
        in_specs=[
            pl.BlockSpec((1, RBLK, D), lambda c, i: (c, i, 0)),
            pl.BlockSpec((1, RBLK, 1), lambda c, i: (c, i, 0)),
            pl.BlockSpec((1, D, D), lambda c, i: (c, 0, 0)),
            pl.BlockSpec((1, 1, D), lambda c, i: (c, 0, 0)),
        ],
        out_specs=pl.BlockSpec((1, RBLK, D), lambda c, i: (c, i, 0)),
        out_shape=jax.ShapeDtypeStruct((NC, VPAD, D), jnp.float32),
    )(accV, dvs, W2, b2)


def _tc_posthead(accV, dvs, vW, vb):
    def body(a_ref, d_ref, w_ref, b_ref, o_ref):
        t = jnp.maximum(a_ref[0] * d_ref[0], 0.0)
        o_ref[0] = _mxu(t, w_ref[0]) + b_ref[0]

    grid = (NC, VPAD // RBLK)
    return pl.pallas_call(
        body,
        grid=grid,
        in_specs=[
            pl.BlockSpec((1, RBLK, D), lambda c, i: (c, i, 0)),
            pl.BlockSpec((1, RBLK, 1), lambda c, i: (c, i, 0)),
            pl.BlockSpec((1, D, 1), lambda c, i: (c, 0, 0)),
            pl.BlockSpec((1, 1, 1), lambda c, i: (c, 0, 0)),
        ],
        out_specs=pl.BlockSpec((1, RBLK, 1), lambda c, i: (c, i, 0)),
        out_shape=jax.ShapeDtypeStruct((NC, VPAD, 1), jnp.float32),
    )(accV, dvs, vW, vb)


def _tc_final(v, mWp, mb):
    def body(v_ref, m_ref, b_ref, o_ref):
        s = jnp.sum(v_ref[...] * m_ref[...]) + b_ref[0, 0]
        o_ref[...] = jnp.full((1, 1), s, jnp.float32)

    return pl.pallas_call(
        body,
        out_shape=jax.ShapeDtypeStruct((1, 1), jnp.float32),
    )(v, mWp, mb)


def kernel(X, v_idx0, e_idx0, v_idx1, e_idx1,
           W1a, b1a, W2a, b2a, vWa, vba,
           W1b, b1b, W2b, b2b, vWb, vbb,
           mW, mb):
    Xp = jnp.pad(X, ((0, VPAD - N_V), (0, 0)))
    sent_v = jnp.full((P_PAD - N_P,), N_V, jnp.int32)
    sent_e = jnp.full((P_PAD - N_P,), N_E, jnp.int32)
    vflat = jnp.concatenate([v_idx0.astype(jnp.int32), sent_v,
                             v_idx1.astype(jnp.int32), sent_v])
    eflat = jnp.concatenate([e_idx0.astype(jnp.int32), sent_e,
                             e_idx1.astype(jnp.int32), sent_e])
    W1 = jnp.stack([W1a, W1b])
    b1 = jnp.stack([b1a, b1b]).reshape(NC, 1, D)
    W2 = jnp.stack([W2a, W2b])
    b2 = jnp.stack([b2a, b2b]).reshape(NC, 1, D)
    vW = jnp.stack([vWa, vWb])
    vb = jnp.stack([vba, vbb]).reshape(NC, 1, 1)
    mWp = jnp.stack([
        jnp.pad(mW[:N_V, 0], (0, VPAD - N_V)),
        jnp.pad(mW[N_V:, 0], (0, VPAD - N_V)),
    ]).reshape(NC, VPAD, 1)

    v3e = vflat.reshape(NC * NS, NCH, 128)
    e3e = eflat.reshape(NC * NS, NCH, 128)
    v3v = vflat.reshape(NC * NS, 2 * NCH, 64)
    e3v = eflat.reshape(NC * NS, 2 * NCH, 64)

    dvp, dep = _sc_degrees(vflat, eflat)
    dvs, dei = _tc_deg(dvp.reshape(NC * NS, VPAD), dep.reshape(NC * NS, EPAD))

    h = _tc_pre(Xp, W1, b1, dvs)
    for layer in range(2):
        accE = _sc_pass_e(h, v3e, e3e)
        Ye = _tc_mid(accE, dei)
        accV = _sc_pass_v(Ye, e3v, v3v)
        if layer == 0:
            h = _tc_postpre(accV, dvs, W2, b2)
    v = _tc_posthead(accV, dvs, vW, vb)
    out = _tc_final(v, mWp, mb.reshape(1, 1))
    return out.reshape(1)


# speedup vs baseline: 1.3981x; 1.3981x over previous
import functools

import jax
import jax.numpy as jnp
from jax import lax
from jax.experimental import pallas as pl
from jax.experimental.pallas import tpu as pltpu
from jax.experimental.pallas import tpu_sc as plsc

N_V = 10000
N_E = 2500
N_P = 320000
D = 128
NC = 2
NS = 16
L = 16
VPAD = 10240
EPAD = 2560
RBLK = 1280
NCH = 160
P_T = NCH * 128
P_PAD = NS * P_T

_mesh = lambda: plsc.VectorSubcoreMesh(core_axis_name="c", subcore_axis_name="s")
_SC_PARAMS = pltpu.CompilerParams(needs_layout_passes=False)


def _sc_degrees(vflat, eflat):
    @functools.partial(
        pl.kernel,
        mesh=_mesh(),
        compiler_params=_SC_PARAMS,
        out_type=(jax.ShapeDtypeStruct((NC * NS * VPAD,), jnp.float32),
                  jax.ShapeDtypeStruct((NC * NS * EPAD,), jnp.float32)),
        scratch_types=[
            pltpu.VMEM((P_T,), jnp.int32),
            pltpu.VMEM((P_T,), jnp.int32),
            pltpu.VMEM((VPAD,), jnp.float32),
            pltpu.VMEM((EPAD,), jnp.float32),
        ],
    )
    def k(vidx_hbm, eidx_hbm, dv_out, de_out, vbuf, ebuf, dvh, deh):
        c = lax.axis_index("c")
        s = lax.axis_index("s")
        wid = c * NS + s
        zero = jnp.zeros((L,), jnp.float32)

        def zloop(ref, n):
            def zbody(j, carry):
                ref[pl.ds(j * L, L)] = zero
                return carry
            lax.fori_loop(0, n, zbody, 0)

        zloop(dvh, VPAD // L)
        zloop(deh, EPAD // L)
        base = c * P_PAD + s * P_T
        pltpu.sync_copy(vidx_hbm.at[pl.ds(base, P_T)], vbuf)
        pltpu.sync_copy(eidx_hbm.at[pl.ds(base, P_T)], ebuf)
        ones = jnp.ones((L,), jnp.float32)

        def body(i, carry):
            vi = vbuf[pl.ds(i * L, L)]
            ei = ebuf[pl.ds(i * L, L)]
            plsc.addupdate_scatter(dvh, [vi], ones)
            plsc.addupdate_scatter(deh, [ei], ones)
            return carry

        lax.fori_loop(0, P_T // L, body, 0)
        pltpu.sync_copy(dvh, dv_out.at[pl.ds(wid * VPAD, VPAD)])
        pltpu.sync_copy(deh, de_out.at[pl.ds(wid * EPAD, EPAD)])

    return k(vflat, eflat)


def _make_sc_pass(trows, arows, chunk, nsec):
    tpad = trows
    nch = P_T // chunk
    spc = nch // nsec
    rpt = arows // NS
    tpt = tpad // NS

    @functools.partial(
        pl.kernel,
        mesh=_mesh(),
        compiler_params=_SC_PARAMS,
        out_type=jax.ShapeDtypeStruct((NC, arows, D), jnp.float32),
        scratch_types=[
            pltpu.VMEM_SHARED((tpad, D), jnp.float32),
            pltpu.VMEM_SHARED((arows, D), jnp.float32),
            pltpu.VMEM((spc, chunk), jnp.int32),
            pltpu.VMEM((spc, chunk), jnp.int32),
            pltpu.VMEM((chunk, D), jnp.float32),
            pltpu.VMEM((16, D), jnp.float32),
            pltpu.SemaphoreType.DMA,
        ],
    )
    def k(table_hbm, gidx_hbm, sidx_hbm, out_hbm,
          tbl, acc, gsec, ssec, rows, zbuf, sz):
        c = lax.axis_index("c")
        s = lax.axis_index("s")
        wid = c * NS + s
        zero = jnp.zeros((L,), jnp.float32)
        for j in range(16):
            for kk in range(D // L):
                zbuf[j, pl.ds(kk * L, L)] = zero
        r0 = s * rpt
        t0 = s * tpt

        def zfire(r, carry):
            pltpu.async_copy(zbuf, acc.at[pl.ds(r0 + r * 16, 16)], sz)
            return carry

        def zdrain(r, carry):
            pltpu.make_async_copy(zbuf, acc.at[pl.ds(r0 + r * 16, 16)], sz).wait()
            return carry

        lax.fori_loop(0, rpt // 16, zfire, 0)
        pltpu.sync_copy(table_hbm.at[c, pl.ds(t0, tpt)], tbl.at[pl.ds(t0, tpt)])
        lax.fori_loop(0, rpt // 16, zdrain, 0)
        plsc.subcore_barrier()

        for sec in range(nsec):
            pltpu.sync_copy(gidx_hbm.at[wid, pl.ds(sec * spc, spc)], gsec)
            pltpu.sync_copy(sidx_hbm.at[wid, pl.ds(sec * spc, spc)], ssec)

            def chunkfn(j, carry):
                pltpu.sync_copy(tbl.at[gsec.at[j]], rows)
                pltpu.sync_copy(rows, acc.at[ssec.at[j]], add=True)
                return carry

            lax.fori_loop(0, spc, chunkfn, 0)

        plsc.subcore_barrier()
        pltpu.sync_copy(acc.at[pl.ds(r0, rpt)], out_hbm.at[c, pl.ds(r0, rpt)])

    return k


_sc_pass_e = _make_sc_pass(VPAD, EPAD, 64, 8)
_sc_pass_v = _make_sc_pass(EPAD, VPAD, 64, 8)


def _mxu(a, b):
    return jnp.dot(a.astype(jnp.bfloat16), b.astype(jnp.bfloat16),
                   preferred_element_type=jnp.float32)


def _tc_deg(dvp, dep):
    def body(dvp_ref, dep_ref, dvs_ref, dei_ref):
        for c in range(NC):
            dv = jnp.sum(dvp_ref[c * NS:(c + 1) * NS, :], axis=0)
            de = jnp.sum(dep_ref[c * NS:(c + 1) * NS, :], axis=0)
            dvs_ref[c, :, 0] = jnp.where(dv > 0, 1.0 / jnp.sqrt(jnp.maximum(dv, 1e-12)), 0.0)
            dei_ref[c, :, 0] = jnp.where(de > 0, 1.0 / jnp.maximum(de, 1e-12), 0.0)

    return pl.pallas_call(
        body,
        out_shape=(jax.ShapeDtypeStruct((NC, VPAD, 1), jnp.float32),
                   jax.ShapeDtypeStruct((NC, EPAD, 1), jnp.float32)),
    )(dvp, dep)


def _tc_pre(Xp, W1, b1, dvs):
    def body(x_ref, w_ref, b_ref, d_ref, o_ref):
        x = x_ref[...]
        o_ref[0] = (_mxu(x, w_ref[0]) + b_ref[0]) * d_ref[0]

    grid = (NC, VPAD // RBLK)
    return pl.pallas_call(
        body,
        grid=grid,
        in_specs=[
            pl.BlockSpec((RBLK, D), lambda c, i: (i, 0)),
            pl.BlockSpec((1, D, D), lambda c, i: (c, 0, 0)),
            pl.BlockSpec((1, 1, D), lambda c, i: (c, 0, 0)),
            pl.BlockSpec((1, RBLK, 1), lambda c, i: (c, i, 0)),
        ],
        out_specs=pl.BlockSpec((1, RBLK, D), lambda c, i: (c, i, 0)),
        out_shape=jax.ShapeDtypeStruct((NC, VPAD, D), jnp.float32),
    )(Xp, W1, b1, dvs)


def _tc_mid(accE, dei):
    def body(a_ref, d_ref, o_ref):
        o_ref[...] = a_ref[...] * d_ref[...]

    return pl.pallas_call(
        body,
        grid=(NC,),
        in_specs=[
            pl.BlockSpec((1, EPAD, D), lambda c: (c, 0, 0)),
            pl.BlockSpec((1, EPAD, 1), lambda c: (c, 0, 0)),
        ],
        out_specs=pl.BlockSpec((1, EPAD, D), lambda c: (c, 0, 0)),
        out_shape=jax.ShapeDtypeStruct((NC, EPAD, D), jnp.float32),
    )(accE, dei)


def _tc_postpre(accV, dvs, W2, b2):
    def body(a_ref, d_ref, w_ref, b_ref, o_ref):
        t = jnp.maximum(a_ref[0] * d_ref[0], 0.0)
        o_ref[0] = (_mxu(t, w_ref[0]) + b_ref[0]) * d_ref[0]

    grid = (NC, VPAD // RBLK)
    return pl.pallas_call(
        body,
        grid=grid,
        in_specs=[
            pl.BlockSpec((1, RBLK, D), lambda c, i: (c, i, 0)),
            pl.BlockSpec((1, RBLK, 1), lambda c, i: (c, i, 0)),
            pl.BlockSpec((1, D, D), lambda c, i: (c, 0, 0)),
            pl.BlockSpec((1, 1, D), lambda c, i: (c, 0, 0)),
        ],
        out_specs=pl.BlockSpec((1, RBLK, D), lambda c, i: (c, i, 0)),
        out_shape=jax.ShapeDtypeStruct((NC, VPAD, D), jnp.float32),
    )(accV, dvs, W2, b2)


def _tc_posthead(accV, dvs, vW, vb):
    def body(a_ref, d_ref, w_ref, b_ref, o_ref):
        t = jnp.maximum(a_ref[0] * d_ref[0], 0.0)
        o_ref[0] = _mxu(t, w_ref[0]) + b_ref[0]

    grid = (NC, VPAD // RBLK)
    return pl.pallas_call(
        body,
        grid=grid,
        in_specs=[
            pl.BlockSpec((1, RBLK, D), lambda c, i: (c, i, 0)),
            pl.BlockSpec((1, RBLK, 1), lambda c, i: (c, i, 0)),
            pl.BlockSpec((1, D, 1), lambda c, i: (c, 0, 0)),
            pl.BlockSpec((1, 1, 1), lambda c, i: (c, 0, 0)),
        ],
        out_specs=pl.BlockSpec((1, RBLK, 1), lambda c, i: (c, i, 0)),
        out_shape=jax.ShapeDtypeStruct((NC, VPAD, 1), jnp.float32),
    )(accV, dvs, vW, vb)


def _tc_final(v, mWp, mb):
    def body(v_ref, m_ref, b_ref, o_ref):
        s = jnp.sum(v_ref[...] * m_ref[...]) + b_ref[0, 0]
        o_ref[...] = jnp.full((1, 1), s, jnp.float32)

    return pl.pallas_call(
        body,
        out_shape=jax.ShapeDtypeStruct((1, 1), jnp.float32),
    )(v, mWp, mb)


def kernel(X, v_idx0, e_idx0, v_idx1, e_idx1,
           W1a, b1a, W2a, b2a, vWa, vba,
           W1b, b1b, W2b, b2b, vWb, vbb,
           mW, mb):
    Xp = jnp.pad(X, ((0, VPAD - N_V), (0, 0)))
    sent_v = jnp.full((P_PAD - N_P,), N_V, jnp.int32)
    sent_e = jnp.full((P_PAD - N_P,), N_E, jnp.int32)
    vflat = jnp.concatenate([v_idx0.astype(jnp.int32), sent_v,
                             v_idx1.astype(jnp.int32), sent_v])
    eflat = jnp.concatenate([e_idx0.astype(jnp.int32), sent_e,
                             e_idx1.astype(jnp.int32), sent_e])
    W1 = jnp.stack([W1a, W1b])
    b1 = jnp.stack([b1a, b1b]).reshape(NC, 1, D)
    W2 = jnp.stack([W2a, W2b])
    b2 = jnp.stack([b2a, b2b]).reshape(NC, 1, D)
    vW = jnp.stack([vWa, vWb])
    vb = jnp.stack([vba, vbb]).reshape(NC, 1, 1)
    mWp = jnp.stack([
        jnp.pad(mW[:N_V, 0], (0, VPAD - N_V)),
        jnp.pad(mW[N_V:, 0], (0, VPAD - N_V)),
    ]).reshape(NC, VPAD, 1)

    v3d = vflat.reshape(NC * NS, 2 * NCH, 64)
    e3d = eflat.reshape(NC * NS, 2 * NCH, 64)

    dvp, dep = _sc_degrees(vflat, eflat)
    dvs, dei = _tc_deg(dvp.reshape(NC * NS, VPAD), dep.reshape(NC * NS, EPAD))

    h = _tc_pre(Xp, W1, b1, dvs)
    for layer in range(2):
        accE = _sc_pass_e(h, v3d, e3d)
        Ye = _tc_mid(accE, dei)
        accV = _sc_pass_v(Ye, e3d, v3d)
        if layer == 0:
            h = _tc_postpre(accV, dvs, W2, b2)
    v = _tc_posthead(accV, dvs, vW, vb)
    out = _tc_final(v, mWp, mb.reshape(1, 1))
    return out.reshape(1)


# speedup vs baseline: 1.4264x; 1.0202x over previous
import functools

import jax
import jax.numpy as jnp
from jax import lax
from jax.experimental import pallas as pl
from jax.experimental.pallas import tpu as pltpu
from jax.experimental.pallas import tpu_sc as plsc

N_V = 10000
N_E = 2500
N_P = 320000
D = 128
NC = 2
NS = 16
L = 16
VPAD = 10240
EPAD = 2560
RBLK = 1280
NCH = 160
P_T = NCH * 128
P_PAD = NS * P_T

_mesh = lambda: plsc.VectorSubcoreMesh(core_axis_name="c", subcore_axis_name="s")
_SC_PARAMS = pltpu.CompilerParams(needs_layout_passes=False)


def _sc_degrees(vflat, eflat):
    @functools.partial(
        pl.kernel,
        mesh=_mesh(),
        compiler_params=_SC_PARAMS,
        out_type=(jax.ShapeDtypeStruct((NC * NS * VPAD,), jnp.float32),
                  jax.ShapeDtypeStruct((NC * NS * EPAD,), jnp.float32)),
        scratch_types=[
            pltpu.VMEM((P_T,), jnp.int32),
            pltpu.VMEM((P_T,), jnp.int32),
            pltpu.VMEM((VPAD,), jnp.float32),
            pltpu.VMEM((EPAD,), jnp.float32),
        ],
    )
    def k(vidx_hbm, eidx_hbm, dv_out, de_out, vbuf, ebuf, dvh, deh):
        c = lax.axis_index("c")
        s = lax.axis_index("s")
        wid = c * NS + s
        zero = jnp.zeros((L,), jnp.float32)

        def zloop(ref, n):
            def zbody(j, carry):
                ref[pl.ds(j * L, L)] = zero
                return carry
            lax.fori_loop(0, n, zbody, 0)

        zloop(dvh, VPAD // L)
        zloop(deh, EPAD // L)
        base = c * P_PAD + s * P_T
        pltpu.sync_copy(vidx_hbm.at[pl.ds(base, P_T)], vbuf)
        pltpu.sync_copy(eidx_hbm.at[pl.ds(base, P_T)], ebuf)
        ones = jnp.ones((L,), jnp.float32)

        def body(i, carry):
            vi = vbuf[pl.ds(i * L, L)]
            ei = ebuf[pl.ds(i * L, L)]
            plsc.addupdate_scatter(dvh, [vi], ones)
            plsc.addupdate_scatter(deh, [ei], ones)
            return carry

        lax.fori_loop(0, P_T // L, body, 0)
        pltpu.sync_copy(dvh, dv_out.at[pl.ds(wid * VPAD, VPAD)])
        pltpu.sync_copy(deh, de_out.at[pl.ds(wid * EPAD, EPAD)])

    return k(vflat, eflat)


def _make_sc_pass(trows, arows, chunk, nsec):
    tpad = trows
    nch = P_T // chunk
    spc = nch // nsec
    rpt = arows // NS
    tpt = tpad // NS

    @functools.partial(
        pl.kernel,
        mesh=_mesh(),
        compiler_params=_SC_PARAMS,
        out_type=jax.ShapeDtypeStruct((NC, arows, D), jnp.float32),
        scratch_types=[
            pltpu.VMEM_SHARED((tpad, D), jnp.float32),
            pltpu.VMEM_SHARED((arows, D), jnp.float32),
            pltpu.VMEM((spc, chunk), jnp.int32),
            pltpu.VMEM((spc, chunk), jnp.int32),
            pltpu.VMEM((chunk, D), jnp.float32),
            pltpu.VMEM((16, D), jnp.float32),
            pltpu.SemaphoreType.DMA,
        ],
    )
    def k(table_hbm, gidx_hbm, sidx_hbm, out_hbm,
          tbl, acc, gsec, ssec, rows, zbuf, sz):
        c = lax.axis_index("c")
        s = lax.axis_index("s")
        wid = c * NS + s
        zero = jnp.zeros((L,), jnp.float32)
        for j in range(16):
            for kk in range(D // L):
                zbuf[j, pl.ds(kk * L, L)] = zero
        r0 = s * rpt
        t0 = s * tpt

        def zfire(r, carry):
            pltpu.async_copy(zbuf, acc.at[pl.ds(r0 + r * 16, 16)], sz)
            return carry

        def zdrain(r, carry):
            pltpu.make_async_copy(zbuf, acc.at[pl.ds(r0 + r * 16, 16)], sz).wait()
            return carry

        lax.fori_loop(0, rpt // 16, zfire, 0)
        pltpu.sync_copy(table_hbm.at[c, pl.ds(t0, tpt)], tbl.at[pl.ds(t0, tpt)])
        lax.fori_loop(0, rpt // 16, zdrain, 0)
        plsc.subcore_barrier()

        for sec in range(nsec):
            pltpu.sync_copy(gidx_hbm.at[wid, pl.ds(sec * spc, spc)], gsec)
            pltpu.sync_copy(sidx_hbm.at[wid, pl.ds(sec * spc, spc)], ssec)

            def chunkfn(j, carry):
                pltpu.sync_copy(tbl.at[gsec.at[j]], rows)
                pltpu.sync_copy(rows, acc.at[ssec.at[j]], add=True)
                return carry

            lax.fori_loop(0, spc, chunkfn, 0)

        plsc.subcore_barrier()
        pltpu.sync_copy(acc.at[pl.ds(r0, rpt)], out_hbm.at[c, pl.ds(r0, rpt)])

    return k


_sc_pass_e = _make_sc_pass(VPAD, EPAD, 128, 5)
_sc_pass_v = _make_sc_pass(EPAD, VPAD, 128, 5)


def _mxu(a, b):
    return jnp.dot(a.astype(jnp.bfloat16), b.astype(jnp.bfloat16),
                   preferred_element_type=jnp.float32)


def _tc_deg(dvp, dep):
    def body(dvp_ref, dep_ref, dvs_ref, dei_ref):
        for c in range(NC):
            dv = jnp.sum(dvp_ref[c * NS:(c + 1) * NS, :], axis=0)
            de = jnp.sum(dep_ref[c * NS:(c + 1) * NS, :], axis=0)
            dvs_ref[c, :, 0] = jnp.where(dv > 0, 1.0 / jnp.sqrt(jnp.maximum(dv, 1e-12)), 0.0)
            dei_ref[c, :, 0] = jnp.where(de > 0, 1.0 / jnp.maximum(de, 1e-12), 0.0)

    return pl.pallas_call(
        body,
        out_shape=(jax.ShapeDtypeStruct((NC, VPAD, 1), jnp.float32),
                   jax.ShapeDtypeStruct((NC, EPAD, 1), jnp.float32)),
    )(dvp, dep)


def _tc_pre(Xp, W1, b1, dvs):
    def body(x_ref, w_ref, b_ref, d_ref, o_ref):
        x = x_ref[...]
        o_ref[0] = (_mxu(x, w_ref[0]) + b_ref[0]) * d_ref[0]

    grid = (NC, VPAD // RBLK)
    return pl.pallas_call(
        body,
        grid=grid,
        in_specs=[
            pl.BlockSpec((RBLK, D), lambda c, i: (i, 0)),
            pl.BlockSpec((1, D, D), lambda c, i: (c, 0, 0)),
            pl.BlockSpec((1, 1, D), lambda c, i: (c, 0, 0)),
            pl.BlockSpec((1, RBLK, 1), lambda c, i: (c, i, 0)),
        ],
        out_specs=pl.BlockSpec((1, RBLK, D), lambda c, i: (c, i, 0)),
        out_shape=jax.ShapeDtypeStruct((NC, VPAD, D), jnp.float32),
    )(Xp, W1, b1, dvs)


def _tc_mid(accE, dei):
    def body(a_ref, d_ref, o_ref):
        o_ref[...] = a_ref[...] * d_ref[...]

    return pl.pallas_call(
        body,
        grid=(NC,),
        in_specs=[
            pl.BlockSpec((1, EPAD, D), lambda c: (c, 0, 0)),
            pl.BlockSpec((1, EPAD, 1), lambda c: (c, 0, 0)),
        ],
        out_specs=pl.BlockSpec((1, EPAD, D), lambda c: (c, 0, 0)),
        out_shape=jax.ShapeDtypeStruct((NC, EPAD, D), jnp.float32),
    )(accE, dei)


def _tc_postpre(accV, dvs, W2, b2):
    def body(a_ref, d_ref, w_ref, b_ref, o_ref):
        t = jnp.maximum(a_ref[0] * d_ref[0], 0.0)
        o_ref[0] = (_mxu(t, w_ref[0]) + b_ref[0]) * d_ref[0]

    grid = (NC, VPAD // RBLK)
    return pl.pallas_call(
        body,
        grid=grid,
        in_specs=[
            pl.BlockSpec((1, RBLK, D), lambda c, i: (c, i, 0)),
            pl.BlockSpec((1, RBLK, 1), lambda c, i: (c, i, 0)),
            pl.BlockSpec((1, D, D), lambda c, i: (c, 0, 0)),
            pl.BlockSpec((1, 1, D), lambda c, i: (c, 0, 0)),
        ],
        out_specs=pl.BlockSpec((1, RBLK, D), lambda c, i: (c, i, 0)),
        out_shape=jax.ShapeDtypeStruct((NC, VPAD, D), jnp.float32),
    )(accV, dvs, W2, b2)


def _tc_posthead(accV, dvs, vW, vb):
    def body(a_ref, d_ref, w_ref, b_ref, o_ref):
        t = jnp.maximum(a_ref[0] * d_ref[0], 0.0)
        o_ref[0] = _mxu(t, w_ref[0]) + b_ref[0]

    grid = (NC, VPAD // RBLK)
    return pl.pallas_call(
        body,
        grid=grid,
        in_specs=[
            pl.BlockSpec((1, RBLK, D), lambda c, i: (c, i, 0)),
            pl.BlockSpec((1, RBLK, 1), lambda c, i: (c, i, 0)),
            pl.BlockSpec((1, D, 1), lambda c, i: (c, 0, 0)),
            pl.BlockSpec((1, 1, 1), lambda c, i: (c, 0, 0)),
        ],
        out_specs=pl.BlockSpec((1, RBLK, 1), lambda c, i: (c, i, 0)),
        out_shape=jax.ShapeDtypeStruct((NC, VPAD, 1), jnp.float32),
    )(accV, dvs, vW, vb)


def _tc_final(v, mWp, mb):
    def body(v_ref, m_ref, b_ref, o_ref):
        s = jnp.sum(v_ref[...] * m_ref[...]) + b_ref[0, 0]
        o_ref[...] = jnp.full((1, 1), s, jnp.float32)

    return pl.pallas_call(
        body,
        out_shape=jax.ShapeDtypeStruct((1, 1), jnp.float32),
    )(v, mWp, mb)


def kernel(X, v_idx0, e_idx0, v_idx1, e_idx1,
           W1a, b1a, W2a, b2a, vWa, vba,
           W1b, b1b, W2b, b2b, vWb, vbb,
           mW, mb):
    Xp = jnp.pad(X, ((0, VPAD - N_V), (0, 0)))
    sent_v = jnp.full((P_PAD - N_P,), N_V, jnp.int32)
    sent_e = jnp.full((P_PAD - N_P,), N_E, jnp.int32)
    vflat = jnp.concatenate([v_idx0.astype(jnp.int32), sent_v,
                             v_idx1.astype(jnp.int32), sent_v])
    eflat = jnp.concatenate([e_idx0.astype(jnp.int32), sent_e,
                             e_idx1.astype(jnp.int32), sent_e])
    W1 = jnp.stack([W1a, W1b])
    b1 = jnp.stack([b1a, b1b]).reshape(NC, 1, D)
    W2 = jnp.stack([W2a, W2b])
    b2 = jnp.stack([b2a, b2b]).reshape(NC, 1, D)
    vW = jnp.stack([vWa, vWb])
    vb = jnp.stack([vba, vbb]).reshape(NC, 1, 1)
    mWp = jnp.stack([
        jnp.pad(mW[:N_V, 0], (0, VPAD - N_V)),
        jnp.pad(mW[N_V:, 0], (0, VPAD - N_V)),
    ]).reshape(NC, VPAD, 1)

    v3d = vflat.reshape(NC * NS, NCH, 128)
    e3d = eflat.reshape(NC * NS, NCH, 128)

    dvp, dep = _sc_degrees(vflat, eflat)
    dvs, dei = _tc_deg(dvp.reshape(NC * NS, VPAD), dep.reshape(NC * NS, EPAD))

    h = _tc_pre(Xp, W1, b1, dvs)
    for layer in range(2):
        accE = _sc_pass_e(h, v3d, e3d)
        Ye = _tc_mid(accE, dei)
        accV = _sc_pass_v(Ye, e3d, v3d)
        if layer == 0:
            h = _tc_postpre(accV, dvs, W2, b2)
    v = _tc_posthead(accV, dvs, vW, vb)
    out = _tc_final(v, mWp, mb.reshape(1, 1))
    return out.reshape(1)
